# baseline (speedup 1.0000x reference)
import functools

import jax
import jax.numpy as jnp
from jax import lax
from jax.experimental import pallas as pl
from jax.experimental.pallas import tpu as pltpu
from jax.experimental.pallas import tpu_sc as plsc

NUM_E = 1000000
DIM = 64
DW = DIM // 4
TOTAL = 4096 * 50
NW = 32
PER_W = TOTAL // NW
CHUNK = 128
NCHUNK = PER_W // CHUNK
GROUPS = CHUNK // 16

_mesh = plsc.VectorSubcoreMesh(core_axis_name="c", subcore_axis_name="s")


@functools.partial(
    pl.kernel,
    out_type=jax.ShapeDtypeStruct((TOTAL, DIM), jnp.float32),
    mesh=_mesh,
    compiler_params=pltpu.CompilerParams(
        needs_layout_passes=False, use_tc_tiling_on_sc=False),
    scratch_types=[
        pltpu.VMEM((PER_W,), jnp.int32),
        pltpu.VMEM((CHUNK, DW), jnp.int32),
        pltpu.VMEM((CHUNK,), jnp.float32),
        pltpu.VMEM((CHUNK,), jnp.float32),
        pltpu.VMEM((CHUNK, DIM), jnp.float32),
        pltpu.SemaphoreType.DMA,
        pltpu.SemaphoreType.DMA,
        pltpu.SemaphoreType.DMA,
    ],
)
def _emb_lookup(tbl, idx_hbm, sc_hbm, zp_hbm, out_hbm,
                idx_v, rows_v, s_v, zp_v, out_v, sem_r, sem_s, sem_z):
    wid = lax.axis_index("s") * 2 + lax.axis_index("c")
    base = wid * PER_W
    pltpu.sync_copy(idx_hbm.at[pl.ds(base, PER_W)], idx_v)

    lanes = lax.iota(jnp.int32, 16)
    scat_idx = [lanes * 4 + k for k in range(4)]

    def chunk_body(c, carry):
        off = pl.multiple_of(c * CHUNK, CHUNK)
        idx_sl = idx_v.at[pl.ds(off, CHUNK)]
        cp_r = pltpu.async_copy(tbl.at[idx_sl], rows_v, sem_r)
        cp_s = pltpu.async_copy(sc_hbm.at[idx_sl], s_v, sem_s)
        cp_z = pltpu.async_copy(zp_hbm.at[idx_sl], zp_v, sem_z)
        cp_r.wait()
        cp_s.wait()
        cp_z.wait()

        def group_body(g, carry2):
            rbase = g * 16
            for j in range(16):
                r = rbase + j
                ridx = jnp.full((16,), r, jnp.int32)
                w = rows_v[r, :]
                sb = plsc.load_gather(s_v, [ridx])
                zb = plsc.load_gather(zp_v, [ridx])
                zs = zb * sb
                for k in range(4):
                    b = (w >> (8 * k)) & 0xFF if k else w & 0xFF
                    v = b.astype(jnp.float32) * sb - zs
                    plsc.store_scatter(out_v, [ridx, scat_idx[k]], v)
            return carry2

        lax.fori_loop(0, GROUPS, group_body, 0)
        pltpu.sync_copy(out_v, out_hbm.at[pl.ds(base + off, CHUNK)])
        return carry

    lax.fori_loop(0, NCHUNK, chunk_body, 0)


def kernel(indices, qweight, scales, zero_points):
    b, h = indices.shape
    n, d = qweight.shape
    idx_flat = indices.reshape(b * h)
    tbl = lax.bitcast_convert_type(qweight.reshape(n, d // 4, 4), jnp.int32)
    out = _emb_lookup(tbl, idx_flat, scales, zero_points)
    return out.reshape(b, h, d)


# speedup vs baseline: 1.6445x; 1.6445x over previous
import functools

import jax
import jax.numpy as jnp
from jax import lax
from jax.experimental import pallas as pl
from jax.experimental.pallas import tpu as pltpu
from jax.experimental.pallas import tpu_sc as plsc

BATCH = 4096
HIST = 50
DIM = 64
TOTAL = BATCH * HIST
NW = 32
PER_W = TOTAL // NW
ROWS_W = PER_W // HIST
CHUNK_B = 8
CHUNK = CHUNK_B * HIST
GSZ = 80
NG = CHUNK // GSZ
NCHUNK = PER_W // CHUNK
GROUPS = CHUNK // 16

_mesh = plsc.VectorSubcoreMesh(core_axis_name="c", subcore_axis_name="s")


@functools.partial(
    pl.kernel,
    out_type=jax.ShapeDtypeStruct((BATCH, HIST, DIM), jnp.float32),
    mesh=_mesh,
    compiler_params=pltpu.CompilerParams(
        needs_layout_passes=False, use_tc_tiling_on_sc=False),
    scratch_types=[
        pltpu.VMEM((PER_W,), jnp.int32),
        pltpu.VMEM((CHUNK, DIM), jnp.uint8),
        pltpu.VMEM((CHUNK,), jnp.float32),
        pltpu.VMEM((CHUNK,), jnp.float32),
        pltpu.VMEM((CHUNK, DIM), jnp.float32),
        pltpu.SemaphoreType.DMA,
        pltpu.SemaphoreType.DMA,
        pltpu.SemaphoreType.DMA,
        pltpu.SemaphoreType.DMA,
    ],
)
def _emb_lookup(tbl, idx_hbm, sc_hbm, zp_hbm, out_hbm,
                idx_v, rows_v, s_v, zp_v, out_v, sem_r, sem_s, sem_z, sem_o):
    wid = lax.axis_index("s") * 2 + lax.axis_index("c")
    base = wid * PER_W
    pltpu.sync_copy(idx_hbm.at[pl.ds(base, PER_W)], idx_v)

    lanes = lax.iota(jnp.int32, 16)
    scat_idx = [lanes * 4 + k for k in range(4)]

    def chunk_body(c, carry):
        off = pl.multiple_of(c * CHUNK, CHUNK)
        for g in range(NG):
            idx_sl = idx_v.at[pl.ds(off + g * GSZ, GSZ)]
            pltpu.async_copy(tbl.at[idx_sl], rows_v.at[pl.ds(g * GSZ, GSZ)],
                             sem_r)
            pltpu.async_copy(sc_hbm.at[idx_sl], s_v.at[pl.ds(g * GSZ, GSZ)],
                             sem_s)
            pltpu.async_copy(zp_hbm.at[idx_sl], zp_v.at[pl.ds(g * GSZ, GSZ)],
                             sem_z)
        pltpu.make_async_copy(tbl.at[idx_v.at[pl.ds(off, CHUNK)]], rows_v,
                              sem_r).wait()
        pltpu.make_async_copy(sc_hbm.at[idx_v.at[pl.ds(off, CHUNK)]], s_v,
                              sem_s).wait()
        pltpu.make_async_copy(zp_hbm.at[idx_v.at[pl.ds(off, CHUNK)]], zp_v,
                              sem_z).wait()

        def group_body(g, carry2):
            rbase = g * 16
            for j in range(16):
                r = rbase + j
                ridx = jnp.full((16,), r, jnp.int32)
                w = plsc.bitcast(rows_v[r, :], jnp.int32)
                sb = plsc.load_gather(s_v, [ridx])
                zb = plsc.load_gather(zp_v, [ridx])
                zs = zb * sb
                for k in range(4):
                    if k == 0:
                        b = w & 0xFF
                    elif k == 3:
                        b = lax.shift_right_logical(w, 24)
                    else:
                        b = (w >> (8 * k)) & 0xFF
                    v = b.astype(jnp.float32) * sb - zs
                    plsc.store_scatter(out_v, [ridx, scat_idx[k]], v)
            return carry2

        lax.fori_loop(0, GROUPS, group_body, 0)
        brow = wid * ROWS_W + c * CHUNK_B
        for b in range(CHUNK_B):
            pltpu.async_copy(out_v.at[pl.ds(b * HIST, HIST)],
                             out_hbm.at[brow + b], sem_o)
        for b in range(CHUNK_B):
            pltpu.make_async_copy(out_v.at[pl.ds(b * HIST, HIST)],
                                  out_hbm.at[brow + b], sem_o).wait()
        return carry

    lax.fori_loop(0, NCHUNK, chunk_body, 0)


def kernel(indices, qweight, scales, zero_points):
    b, h = indices.shape
    idx_flat = indices.reshape(b * h)
    return _emb_lookup(qweight, idx_flat, scales, zero_points)
